# baseline (speedup 1.0000x reference)
import functools

import jax
import jax.numpy as jnp
from jax.experimental import pallas as pl
from jax.experimental.pallas import tpu as pltpu

_STEPS = 3


def _body(rep_ref, maskf_ref, wih_ref, whh_ref, b_ref, wout_ref, bout_ref,
          y_ref):
    rep = rep_ref[...]
    rep_t = jnp.swapaxes(rep, 1, 2)
    maskf = maskf_ref[...]
    bb, n, d = rep.shape
    q_star = jnp.zeros((bb, 2 * d), jnp.float32)
    h = jnp.zeros((bb, d), jnp.float32)
    c = jnp.zeros((bb, d), jnp.float32)
    bias = b_ref[...]
    for _ in range(_STEPS):
        gates = (jnp.dot(q_star, wih_ref[...],
                         preferred_element_type=jnp.float32)
                 + jnp.dot(h, whh_ref[...],
                           preferred_element_type=jnp.float32)
                 + bias)
        gi = jax.nn.sigmoid(gates[:, 0 * d:1 * d])
        gf = jax.nn.sigmoid(gates[:, 1 * d:2 * d])
        gg = jnp.tanh(gates[:, 2 * d:3 * d])
        go = jax.nn.sigmoid(gates[:, 3 * d:4 * d])
        c = gf * c + gi * gg
        h = go * jnp.tanh(c)
        e = jax.lax.dot_general(
            h, rep_t, (((1,), (1,)), ((0,), (0,))),
            preferred_element_type=jnp.float32)
        e = jnp.where(maskf > 0, e, -jnp.inf)
        e = e - jnp.max(e, axis=1, keepdims=True)
        a = jnp.exp(e) * maskf
        a = a / jnp.sum(a, axis=1, keepdims=True)
        r = jax.lax.dot_general(
            a, rep, (((1,), (1,)), ((0,), (0,))),
            preferred_element_type=jnp.float32)
        q_star = jnp.concatenate([h, r], axis=-1)
    y = jnp.dot(q_star, wout_ref[...],
                preferred_element_type=jnp.float32) + bout_ref[...]
    y_ref[...] = y


@functools.partial(jax.jit, static_argnames=("interpret",))
def kernel(representation, atom_mask, W_ih, W_hh, b_ih, b_hh, W_out, b_out,
           mean, stddev, interpret=False):
    b, n, d = representation.shape
    bb = 8
    maskf = atom_mask.astype(jnp.float32)
    wih_t = W_ih.T
    whh_t = W_hh.T
    bias = (b_ih + b_hh)[None, :]
    wout_t = W_out.T
    bout = b_out[None, :]

    y = pl.pallas_call(
        _body,
        grid=(b // bb,),
        in_specs=[
            pl.BlockSpec((bb, n, d), lambda i: (i, 0, 0)),
            pl.BlockSpec((bb, n), lambda i: (i, 0)),
            pl.BlockSpec(wih_t.shape, lambda i: (0, 0)),
            pl.BlockSpec(whh_t.shape, lambda i: (0, 0)),
            pl.BlockSpec(bias.shape, lambda i: (0, 0)),
            pl.BlockSpec(wout_t.shape, lambda i: (0, 0)),
            pl.BlockSpec(bout.shape, lambda i: (0, 0)),
        ],
        out_specs=pl.BlockSpec((bb, 1), lambda i: (i, 0)),
        out_shape=jax.ShapeDtypeStruct((b, 1), jnp.float32),
        interpret=interpret,
    )(representation, maskf, wih_t, whh_t, bias, wout_t, bout)
    return y * stddev + mean


# speedup vs baseline: 1.3556x; 1.3556x over previous
import functools

import jax
import jax.numpy as jnp
from jax.experimental import pallas as pl
from jax.experimental.pallas import tpu as pltpu

_STEPS = 3


def _body(rep_ref, maskf_ref, wih_ref, whh_ref, b_ref, wout_ref, bout_ref,
          y_ref):
    rep = rep_ref[...]
    rep_t = jnp.swapaxes(rep, 1, 2)
    maskf = maskf_ref[...]
    bb, n, d = rep.shape
    q_star = jnp.zeros((bb, 2 * d), jnp.float32)
    h = jnp.zeros((bb, d), jnp.float32)
    c = jnp.zeros((bb, d), jnp.float32)
    bias = b_ref[...]
    for _ in range(_STEPS):
        gates = (jnp.dot(q_star, wih_ref[...],
                         preferred_element_type=jnp.float32)
                 + jnp.dot(h, whh_ref[...],
                           preferred_element_type=jnp.float32)
                 + bias)
        gi = jax.nn.sigmoid(gates[:, 0 * d:1 * d])
        gf = jax.nn.sigmoid(gates[:, 1 * d:2 * d])
        gg = jnp.tanh(gates[:, 2 * d:3 * d])
        go = jax.nn.sigmoid(gates[:, 3 * d:4 * d])
        c = gf * c + gi * gg
        h = go * jnp.tanh(c)
        e = jax.lax.dot_general(
            h, rep_t, (((1,), (1,)), ((0,), (0,))),
            preferred_element_type=jnp.float32)
        e = jnp.where(maskf > 0, e, -jnp.inf)
        e = e - jnp.max(e, axis=1, keepdims=True)
        a = jnp.exp(e) * maskf
        a = a / jnp.sum(a, axis=1, keepdims=True)
        r = jax.lax.dot_general(
            a, rep, (((1,), (1,)), ((0,), (0,))),
            preferred_element_type=jnp.float32)
        q_star = jnp.concatenate([h, r], axis=-1)
    y = jnp.dot(q_star, wout_ref[...],
                preferred_element_type=jnp.float32) + bout_ref[...]
    y_ref[...] = y


@functools.partial(jax.jit, static_argnames=("interpret",))
def kernel(representation, atom_mask, W_ih, W_hh, b_ih, b_hh, W_out, b_out,
           mean, stddev, interpret=False):
    b, n, d = representation.shape
    bb = 32
    maskf = atom_mask.astype(jnp.float32)
    wih_t = W_ih.T
    whh_t = W_hh.T
    bias = (b_ih + b_hh)[None, :]
    wout_t = W_out.T
    bout = b_out[None, :]

    y = pl.pallas_call(
        _body,
        grid=(b // bb,),
        in_specs=[
            pl.BlockSpec((bb, n, d), lambda i: (i, 0, 0)),
            pl.BlockSpec((bb, n), lambda i: (i, 0)),
            pl.BlockSpec(wih_t.shape, lambda i: (0, 0)),
            pl.BlockSpec(whh_t.shape, lambda i: (0, 0)),
            pl.BlockSpec(bias.shape, lambda i: (0, 0)),
            pl.BlockSpec(wout_t.shape, lambda i: (0, 0)),
            pl.BlockSpec(bout.shape, lambda i: (0, 0)),
        ],
        out_specs=pl.BlockSpec((bb, 1), lambda i: (i, 0)),
        out_shape=jax.ShapeDtypeStruct((b, 1), jnp.float32),
        interpret=interpret,
    )(representation, maskf, wih_t, whh_t, bias, wout_t, bout)
    return y * stddev + mean
